# baseline (speedup 1.0000x reference)
import functools
import jax
import jax.numpy as jnp
from jax import lax
from jax.experimental import pallas as pl
from jax.experimental.pallas import tpu as pltpu
from jax.experimental.pallas import tpu_sc as plsc

_N = 100000
_D = 128
_K = 10
_SPEED = 2.0

_NW = 32
_CHUNK = 256
_CPW = 1
_RPW = _CHUNK * _CPW
_M = _NW * _RPW
_NT = _N - _M

_GRID = 4
_BLK = -(-_NT // (_GRID * 128)) * 128
_G = _BLK // 128
_SROWS = _GRID * _G


def _tc_main_kernel(x_ref, data_ref, cand_ref, d2_ref):
    i = pl.program_id(0)
    xv = x_ref[...]
    blk = data_ref[...]
    t = jnp.swapaxes(blk.reshape(_G, 128, 128), 1, 2)
    xc = xv.reshape(1, 128, 1)
    d2 = jnp.sum(t * (t - 2.0 * xc), axis=1) + jnp.sum(xv * xv)
    g = jax.lax.broadcasted_iota(jnp.int32, d2.shape, 0)
    r = jax.lax.broadcasted_iota(jnp.int32, d2.shape, 1)
    row = i * _BLK + g * 128 + r
    d2 = jnp.where(row < _NT, d2, jnp.inf)
    d2_ref[pl.ds(i * _G, _G), :] = d2

    @pl.when(i == _GRID - 1)
    def _finalize():
        s = d2_ref[...]
        fi = (jax.lax.broadcasted_iota(jnp.int32, s.shape, 0) * 128
              + jax.lax.broadcasted_iota(jnp.int32, s.shape, 1))
        cr = jax.lax.broadcasted_iota(jnp.int32, (8, 128), 0)
        cl = jax.lax.broadcasted_iota(jnp.int32, (8, 128), 1)
        cand = jnp.full((8, 128), jnp.inf, jnp.float32)
        for k in range(_K):
            m = jnp.min(s)
            cand = jnp.where((cr == 0) & (cl == k), m, cand)
            idx = jnp.min(jnp.where(s == m, fi, jnp.int32(2**31 - 1)))
            s = jnp.where(fi == idx, jnp.inf, s)
        cand_ref[...] = cand


def _tc_main(x2, data):
    return pl.pallas_call(
        _tc_main_kernel,
        grid=(_GRID,),
        in_specs=[
            pl.BlockSpec((1, _D), lambda i: (0, 0)),
            pl.BlockSpec((_BLK, _D), lambda i: (i, 0)),
        ],
        out_specs=pl.BlockSpec((8, 128), lambda i: (0, 0)),
        out_shape=jax.ShapeDtypeStruct((8, 128), jnp.float32),
        scratch_shapes=[pltpu.VMEM((_SROWS, 128), jnp.float32)],
    )(x2, data)


def _sc_distances(x, data):
    mesh = plsc.VectorSubcoreMesh(
        core_axis_name="c", subcore_axis_name="s", num_cores=2)

    @functools.partial(
        pl.kernel, mesh=mesh,
        out_type=jax.ShapeDtypeStruct((_M,), jnp.float32),
        scratch_types=[
            pltpu.VMEM((_D,), jnp.float32),
            pltpu.VMEM((_CHUNK, _D), jnp.float32),
            pltpu.VMEM((_CHUNK, _D), jnp.float32),
            pltpu.VMEM((_RPW,), jnp.float32),
            pltpu.SemaphoreType.DMA,
            pltpu.SemaphoreType.DMA,
        ],
    )
    def k(x_hbm, data_hbm, out_hbm, x_v, buf_a, buf_b, d2buf,
          sem_a, sem_b):
        wid = lax.axis_index("s") * 2 + lax.axis_index("c")
        w0 = _NT + wid * _RPW
        pltpu.sync_copy(x_hbm, x_v)
        xw = [x_v[pl.ds(j * 16, 16)] for j in range(8)]
        li = jax.lax.broadcasted_iota(jnp.int32, (16,), 0)
        dnums = lax.GatherDimensionNumbers(
            offset_dims=(), collapsed_slice_dims=(0,), start_index_map=(0,))
        perms = [(li ^ k)[:, None] for k in (1, 2, 4, 8)]

        def lane_sum(v):
            for p in perms:
                v = v + lax.gather(
                    v, p, dnums, slice_sizes=(1,),
                    mode=lax.GatherScatterMode.PROMISE_IN_BOUNDS)
            return v

        bufs = [buf_a, buf_b]
        sems = [sem_a, sem_b]
        copies = [None, None]
        copies[0] = pltpu.make_async_copy(
            data_hbm.at[pl.ds(w0, _CHUNK)], buf_a, sem_a)
        copies[0].start()

        for c in range(_CPW):
            if c + 1 < _CPW:
                copies[(c + 1) % 2] = pltpu.make_async_copy(
                    data_hbm.at[pl.ds(w0 + (c + 1) * _CHUNK, _CHUNK)],
                    bufs[(c + 1) % 2], sems[(c + 1) % 2])
                copies[(c + 1) % 2].start()
            copies[c % 2].wait()
            buf = bufs[c % 2]

            def group(g, _, buf=buf, c=c):
                base = g * 16
                dv = jnp.zeros((16,), jnp.float32)
                for r in range(16):
                    acc = None
                    for j in range(8):
                        t = buf[base + r, pl.ds(j * 16, 16)]
                        d = t - xw[j]
                        acc = d * d if acc is None else acc + d * d
                    dv = jnp.where(li == r, lane_sum(acc), dv)
                d2buf[pl.ds(c * _CHUNK + base, 16)] = dv
                return 0

            lax.fori_loop(0, _CHUNK // 16, group, 0)

        pltpu.sync_copy(d2buf, out_hbm.at[pl.ds(wid * _RPW, _RPW)])

    return k(x, data)


_MROWS = _M // 128


def _merge_kernel(x_ref, y_ref, cand_ref, d2sc_ref, out_ref):
    s = jnp.concatenate([cand_ref[...], d2sc_ref[...]], axis=0)
    fi = (jax.lax.broadcasted_iota(jnp.int32, s.shape, 0) * 128
          + jax.lax.broadcasted_iota(jnp.int32, s.shape, 1))
    total = jnp.float32(0.0)
    for _ in range(_K):
        m = jnp.min(s)
        total = total + jnp.sqrt(m)
        idx = jnp.min(jnp.where(s == m, fi, jnp.int32(2**31 - 1)))
        s = jnp.where(fi == idx, jnp.inf, s)
    xy = x_ref[...] - y_ref[...]
    geo = jnp.sqrt(jnp.sum(xy * xy)) / jnp.float32(_SPEED)
    out_ref[...] = (geo + total / jnp.float32(_K)).reshape(1, 1)


def _merge(x2, y2, cand, d2sc):
    return pl.pallas_call(
        _merge_kernel,
        in_specs=[
            pl.BlockSpec((1, _D), lambda: (0, 0)),
            pl.BlockSpec((1, _D), lambda: (0, 0)),
            pl.BlockSpec((8, 128), lambda: (0, 0)),
            pl.BlockSpec((_MROWS, 128), lambda: (0, 0)),
        ],
        out_specs=pl.BlockSpec((1, 1), lambda: (0, 0)),
        out_shape=jax.ShapeDtypeStruct((1, 1), jnp.float32),
    )(x2, y2, cand, d2sc)


@jax.jit
def kernel(x, y, data):
    x2 = x.reshape(1, _D)
    y2 = y.reshape(1, _D)
    cand = _tc_main(x2, data)
    d2sc = _sc_distances(x, data)
    out = _merge(x2, y2, cand, d2sc.reshape(_MROWS, 128))
    return out[0, 0]


# speedup vs baseline: 1.2910x; 1.2910x over previous
import jax
import jax.numpy as jnp
from jax.experimental import pallas as pl
from jax.experimental.pallas import tpu as pltpu

_N = 100000
_D = 128
_K = 10
_SPEED = 2.0

_GRID = 4
_BLK = 25088
_G = _BLK // 128


def _extract_topk(s, k):
    fi = (jax.lax.broadcasted_iota(jnp.int32, s.shape, 0) * 128
          + jax.lax.broadcasted_iota(jnp.int32, s.shape, 1))
    cl = jax.lax.broadcasted_iota(jnp.int32, (1, 128), 1)
    cand = jnp.full((1, 128), jnp.inf, jnp.float32)
    for t in range(k):
        m = jnp.min(s)
        cand = jnp.where(cl == t, m, cand)
        idx = jnp.min(jnp.where(s == m, fi, jnp.int32(2**31 - 1)))
        s = jnp.where(fi == idx, jnp.inf, s)
    return cand


def _dist_topk_kernel(x_ref, y_ref, data_ref, out_ref, cands_ref):
    i = pl.program_id(0)
    xv = x_ref[...]
    blk = data_ref[...]
    t = jnp.swapaxes(blk.reshape(_G, 128, 128), 1, 2)
    xc = xv.reshape(1, 128, 1)
    d2 = jnp.sum(t * (t - 2.0 * xc), axis=1) + jnp.sum(xv * xv)

    g = jax.lax.broadcasted_iota(jnp.int32, d2.shape, 0)
    r = jax.lax.broadcasted_iota(jnp.int32, d2.shape, 1)
    row = i * _BLK + g * 128 + r
    d2 = jnp.where(row < _N, d2, jnp.inf)

    @pl.when(i == 0)
    def _init():
        cands_ref[...] = jnp.full((8, 128), jnp.inf, jnp.float32)

    cands_ref[pl.ds(i, 1), :] = _extract_topk(d2, _K)

    @pl.when(i == _GRID - 1)
    def _finalize():
        cand = _extract_topk(cands_ref[...], _K)
        lane = jax.lax.broadcasted_iota(jnp.int32, (1, 128), 1)
        vals = jnp.where(lane < _K, jnp.sqrt(cand), 0.0)
        xy = x_ref[...] - y_ref[...]
        geo = jnp.sqrt(jnp.sum(xy * xy)) / jnp.float32(_SPEED)
        out_ref[...] = (geo + jnp.sum(vals) / jnp.float32(_K)).reshape(1, 1)


@jax.jit
def kernel(x, y, data):
    x2 = x.reshape(1, _D)
    y2 = y.reshape(1, _D)
    out = pl.pallas_call(
        _dist_topk_kernel,
        grid=(_GRID,),
        in_specs=[
            pl.BlockSpec((1, _D), lambda i: (0, 0)),
            pl.BlockSpec((1, _D), lambda i: (0, 0)),
            pl.BlockSpec((_BLK, _D), lambda i: (i, 0)),
        ],
        out_specs=pl.BlockSpec((1, 1), lambda i: (0, 0)),
        out_shape=jax.ShapeDtypeStruct((1, 1), jnp.float32),
        scratch_shapes=[pltpu.VMEM((8, 128), jnp.float32)],
    )(x2, y2, data)
    return out[0, 0]


# speedup vs baseline: 1.7925x; 1.3884x over previous
import jax
import jax.numpy as jnp
from jax.experimental import pallas as pl
from jax.experimental.pallas import tpu as pltpu

_N = 100000
_D = 128
_K = 10
_SPEED = 2.0

_BLK = 25088
_GRID = (_N + _BLK - 1) // _BLK
_SROWS = _GRID * (_BLK // 128)


def _dist_topk_kernel(x_ref, y_ref, data_ref, out_ref, d2_ref):
    i = pl.program_id(0)
    xv = x_ref[...]
    blk = data_ref[...]
    diff = blk - xv
    sq = diff * diff
    d2 = jnp.sum(sq.reshape(_BLK // 128, 128, 128), axis=2)

    g = jax.lax.broadcasted_iota(jnp.int32, d2.shape, 0)
    r = jax.lax.broadcasted_iota(jnp.int32, d2.shape, 1)
    row = i * _BLK + g * 128 + r
    d2 = jnp.where(row < _N, d2, jnp.inf)
    d2_ref[pl.ds(i * (_BLK // 128), _BLK // 128), :] = d2

    @pl.when(i == _GRID - 1)
    def _finalize():
        s = d2_ref[...]
        fi = (jax.lax.broadcasted_iota(jnp.int32, s.shape, 0) * 128
              + jax.lax.broadcasted_iota(jnp.int32, s.shape, 1))
        total = jnp.float32(0.0)
        for _ in range(_K):
            m = jnp.min(s)
            total = total + jnp.sqrt(m)
            idx = jnp.min(jnp.where(s == m, fi, jnp.int32(2**31 - 1)))
            s = jnp.where(fi == idx, jnp.inf, s)
        xy = x_ref[...] - y_ref[...]
        geo = jnp.sqrt(jnp.sum(xy * xy)) / jnp.float32(_SPEED)
        out_ref[...] = (geo + total / jnp.float32(_K)).reshape(1, 1)


@jax.jit
def kernel(x, y, data):
    x2 = x.reshape(1, _D)
    y2 = y.reshape(1, _D)
    out = pl.pallas_call(
        _dist_topk_kernel,
        grid=(_GRID,),
        in_specs=[
            pl.BlockSpec((1, _D), lambda i: (0, 0)),
            pl.BlockSpec((1, _D), lambda i: (0, 0)),
            pl.BlockSpec((_BLK, _D), lambda i: (i, 0)),
        ],
        out_specs=pl.BlockSpec((1, 1), lambda i: (0, 0)),
        out_shape=jax.ShapeDtypeStruct((1, 1), jnp.float32),
        scratch_shapes=[pltpu.VMEM((_SROWS, 128), jnp.float32)],
    )(x2, y2, data)
    return out[0, 0]


# speedup vs baseline: 1.9456x; 1.0855x over previous
import jax
import jax.numpy as jnp
from jax.experimental import pallas as pl
from jax.experimental.pallas import tpu as pltpu

_N = 100000
_D = 128
_K = 10
_SPEED = 2.0

_BLK = 25088
_GRID = (_N + _BLK - 1) // _BLK
_SROWS = _GRID * (_BLK // 128)


def _dist_topk_kernel(x_ref, y_ref, data_ref, out_ref, d2_ref):
    i = pl.program_id(0)
    xv = x_ref[...]
    blk = data_ref[...]
    diff = blk - xv
    sq = diff * diff
    d2 = jnp.sum(sq.reshape(_BLK // 128, 128, 128), axis=2)

    g = jax.lax.broadcasted_iota(jnp.int32, d2.shape, 0)
    r = jax.lax.broadcasted_iota(jnp.int32, d2.shape, 1)
    row = i * _BLK + g * 128 + r
    d2 = jnp.where(row < _N, d2, jnp.inf)
    d2_ref[pl.ds(i * (_BLK // 128), _BLK // 128), :] = d2

    @pl.when(i == _GRID - 1)
    def _finalize():
        s = d2_ref[...]
        total = jnp.float32(0.0)
        rem = jnp.float32(_K)
        for _ in range(_K):
            m = jnp.min(s)
            eq = s == m
            c = jnp.sum(jnp.where(eq, 1.0, 0.0))
            take = jnp.minimum(c, rem)
            total = total + jnp.where(take > 0, take * jnp.sqrt(m), 0.0)
            rem = rem - take
            s = jnp.where(eq, jnp.inf, s)
        xy = x_ref[...] - y_ref[...]
        geo = jnp.sqrt(jnp.sum(xy * xy)) / jnp.float32(_SPEED)
        out_ref[...] = (geo + total / jnp.float32(_K)).reshape(1, 1)


@jax.jit
def kernel(x, y, data):
    x2 = x.reshape(1, _D)
    y2 = y.reshape(1, _D)
    out = pl.pallas_call(
        _dist_topk_kernel,
        grid=(_GRID,),
        in_specs=[
            pl.BlockSpec((1, _D), lambda i: (0, 0)),
            pl.BlockSpec((1, _D), lambda i: (0, 0)),
            pl.BlockSpec((_BLK, _D), lambda i: (i, 0)),
        ],
        out_specs=pl.BlockSpec((1, 1), lambda i: (0, 0)),
        out_shape=jax.ShapeDtypeStruct((1, 1), jnp.float32),
        scratch_shapes=[pltpu.VMEM((_SROWS, 128), jnp.float32)],
    )(x2, y2, data)
    return out[0, 0]
